# Initial kernel scaffold; baseline (speedup 1.0000x reference)
import jax
import jax.numpy as jnp
from jax.experimental import pallas as pl


def kernel(query_points_coords, context_coords, context_features, Wq1, bq1, Wq2, bq2, Wrel, brel, Wqa, bqa, Wka, bka, Wva, bva, Woa, boa, gamma, beta, Wh1, bh1, Wh2, bh2, Wh3, bh3):
    raise NotImplementedError("write your pallas kernel here")


# speedup vs baseline: 1.4679x; 1.4679x over previous
import functools

import jax
import jax.numpy as jnp
from jax import lax
from jax.experimental import pallas as pl
from jax.experimental.pallas import tpu as pltpu
from jax.experimental.pallas import tpu_sc as plsc

K = 16
H = 4
BQ = 256
CCH = 2048
NPAD = 51200
NROWPAD = 50176
BIG_I32 = 2147483647
INF_F32 = float("inf")


def _prep_body(f_ref, c_ref, w_ref, o_ref):
    cx = c_ref[:, 0:1]
    cy = c_ref[:, 1:2]
    cz = c_ref[:, 2:3]
    proj = cx * w_ref[0:1, :] + cy * w_ref[1:2, :] + cz * w_ref[2:3, :]
    o_ref[...] = f_ref[...] - proj


def _prep(feats_pad, coords_pad, Wrel):
    n = feats_pad.shape[0]
    d = feats_pad.shape[1]
    nb = n // 512
    return pl.pallas_call(
        _prep_body,
        grid=(nb,),
        in_specs=[
            pl.BlockSpec((512, d), lambda i: (i, 0)),
            pl.BlockSpec((512, 3), lambda i: (i, 0)),
            pl.BlockSpec((3, d), lambda i: (0, 0)),
        ],
        out_specs=pl.BlockSpec((512, d), lambda i: (i, 0)),
        out_shape=jax.ShapeDtypeStruct((n, d), jnp.float32),
    )(feats_pad, coords_pad, Wrel)


def _extract_topk(vals, pos, k):
    out_v, out_i = [], []
    for _ in range(k):
        m = jnp.min(vals, axis=1, keepdims=True)
        pm = jnp.min(jnp.where(vals == m, pos, BIG_I32), axis=1, keepdims=True)
        out_v.append(m)
        out_i.append(pm)
        vals = jnp.where(pos == pm, INF_F32, vals)
    return jnp.concatenate(out_v, axis=1), jnp.concatenate(out_i, axis=1)


def _knn_body(q_ref, ct_ref, o_ref, bval, bidx):
    j = pl.program_id(1)

    @pl.when(j == 0)
    def _():
        bval[...] = jnp.full((BQ, K), INF_F32, jnp.float32)
        bidx[...] = jnp.full((BQ, K), BIG_I32, jnp.int32)

    cx = ct_ref[0:1, :]
    cy = ct_ref[1:2, :]
    cz = ct_ref[2:3, :]
    qx = q_ref[:, 0:1]
    qy = q_ref[:, 1:2]
    qz = q_ref[:, 2:3]
    c2 = cx * cx + cy * cy + cz * cz
    q2 = qx * qx + qy * qy + qz * qz
    dot = jnp.dot(q_ref[...], ct_ref[0:3, :],
                  preferred_element_type=jnp.float32)
    d2 = jnp.sqrt(jnp.maximum(q2 + c2 - 2.0 * dot, 0.0))
    pos = lax.broadcasted_iota(jnp.int32, (BQ, CCH), 1) + j * CCH

    nv, ni = _extract_topk(d2, pos, K)

    cand_v = jnp.concatenate([bval[...], nv], axis=1)
    cand_i = jnp.concatenate([bidx[...], ni], axis=1)
    mv, mi = _extract_topk(cand_v, cand_i, K)
    bval[...] = mv
    bidx[...] = mi

    @pl.when(j == NPAD // CCH - 1)
    def _():
        o_ref[...] = bidx[...]


def _knn(q, ct):
    nq = q.shape[0]
    return pl.pallas_call(
        _knn_body,
        grid=(nq // BQ, NPAD // CCH),
        in_specs=[
            pl.BlockSpec((BQ, 3), lambda i, j: (i, 0)),
            pl.BlockSpec((8, CCH), lambda i, j: (0, j)),
        ],
        out_specs=pl.BlockSpec((BQ, K), lambda i, j: (i, 0)),
        out_shape=jax.ShapeDtypeStruct((nq, K), jnp.int32),
        scratch_shapes=[
            pltpu.VMEM((BQ, K), jnp.float32),
            pltpu.VMEM((BQ, K), jnp.int32),
        ],
    )(q, ct)


NC_SC = 2
NS_SC = 16
NW_SC = NC_SC * NS_SC


def _gather(idx_flat, table):
    b = idx_flat.shape[0]
    d = table.shape[1]
    b_per_w = b // NW_SC
    n_sub = b_per_w // 128
    mesh = plsc.VectorSubcoreMesh(core_axis_name="c", subcore_axis_name="s")

    @functools.partial(
        pl.kernel,
        mesh=mesh,
        out_type=jax.ShapeDtypeStruct((b, d), jnp.float32),
        scratch_types=[
            pltpu.VMEM((b_per_w,), jnp.int32),
            pltpu.VMEM((128, d), jnp.float32),
            pltpu.VMEM((128, d), jnp.float32),
            pltpu.SemaphoreType.DMA,
            pltpu.SemaphoreType.DMA,
        ],
    )
    def gk(idx_hbm, t_hbm, out_hbm, idx_v, rows0, rows1, sem0, sem1):
        wid = lax.axis_index("s") * NC_SC + lax.axis_index("c")
        base = wid * b_per_w
        pltpu.sync_copy(idx_hbm.at[pl.ds(base, b_per_w)], idx_v)
        rows = (rows0, rows1)
        sems = (sem0, sem1)
        cps = [None, None]
        for c in range(n_sub):
            s = c % 2
            cps[s] = pltpu.async_copy(
                t_hbm.at[idx_v.at[pl.ds(c * 128, 128)]], rows[s], sems[s])
            if c > 0:
                cps[1 - s].wait()
                pltpu.sync_copy(rows[1 - s],
                                out_hbm.at[pl.ds(base + (c - 1) * 128, 128)])
        cps[(n_sub - 1) % 2].wait()
        pltpu.sync_copy(rows[(n_sub - 1) % 2],
                        out_hbm.at[pl.ds(base + (n_sub - 1) * 128, 128)])

    return gk(idx_flat, table)


def _dense_body(q_ref, g_ref, wq1_ref, bq1_ref, wq2_ref, bq2_ref,
                wrel_ref, brel_ref, wqa_ref, bqa_ref, wka_ref, bka_ref,
                wva_ref, bva_ref, woa_ref, boa_ref, gamma_ref, beta_ref,
                wh1_ref, bh1_ref, wh2_ref, bh2_ref, wh3_ref, bh3_ref, o_ref):
    d = 128
    dh = d // H
    qx = q_ref[:, 0:1]
    qy = q_ref[:, 1:2]
    qz = q_ref[:, 2:3]

    def lin3(w_ref, b_ref):
        return (qx * w_ref[0:1, :] + qy * w_ref[1:2, :] + qz * w_ref[2:3, :]
                + b_ref[...])

    def mm(a, w_ref, b_ref):
        r = jnp.dot(a, w_ref[...], preferred_element_type=jnp.float32)
        return r + b_ref[...]

    qe = mm(jax.nn.gelu(lin3(wq1_ref, bq1_ref)), wq2_ref, bq2_ref)
    qrel = lin3(wrel_ref, brel_ref)

    bq = qe.shape[0]
    qrel_rep = jnp.broadcast_to(
        qrel[:, None, :], (bq, K, d)).reshape(bq * K, d)
    kv = g_ref[...] + qrel_rep
    kh = mm(kv, wka_ref, bka_ref)
    vh = mm(kv, wva_ref, bva_ref)
    qh = mm(qe, wqa_ref, bqa_ref)
    qh_rep = jnp.broadcast_to(qh[:, None, :], (bq, K, d)).reshape(bq * K, d)

    sfull = qh_rep * kh
    scale = 1.0 / (dh ** 0.5)
    scores = jnp.concatenate(
        [jnp.sum(sfull[:, h * dh:(h + 1) * dh], axis=1, keepdims=True)
         for h in range(H)], axis=1) * scale
    s3 = scores.reshape(bq, K, H)
    smax = jnp.max(s3, axis=1, keepdims=True)
    e = jnp.exp(s3 - smax)
    attn = e / jnp.sum(e, axis=1, keepdims=True)

    attn_w = jnp.concatenate(
        [jnp.broadcast_to(attn[:, :, h:h + 1], (bq, K, dh)) for h in range(H)],
        axis=2)
    vh3 = vh.reshape(bq, K, d)
    heads = jnp.sum(attn_w * vh3, axis=1)

    agg = mm(heads, woa_ref, boa_ref) + qe
    mu = jnp.mean(agg, axis=-1, keepdims=True)
    var = jnp.mean((agg - mu) * (agg - mu), axis=-1, keepdims=True)
    an = (agg - mu) / jnp.sqrt(var + 1e-5) * gamma_ref[...] + beta_ref[...]

    h1 = jax.nn.gelu(mm(an, wh1_ref, bh1_ref))
    h2 = jax.nn.gelu(mm(h1, wh2_ref, bh2_ref))
    o_ref[...] = mm(h2, wh3_ref, bh3_ref)


def _dense(q, g, weights):
    nq = q.shape[0]
    full = lambda shape: pl.BlockSpec(shape, lambda i: tuple(0 for _ in shape))
    wspecs = [full(w.shape) for w in weights]
    return pl.pallas_call(
        _dense_body,
        grid=(nq // BQ,),
        in_specs=[
            pl.BlockSpec((BQ, 3), lambda i: (i, 0)),
            pl.BlockSpec((BQ * K, 128), lambda i: (i, 0)),
        ] + wspecs,
        out_specs=pl.BlockSpec((BQ, 1), lambda i: (i, 0)),
        out_shape=jax.ShapeDtypeStruct((nq, 1), jnp.float32),
    )(q, g, *weights)


def kernel(query_points_coords, context_coords, context_features,
           Wq1, bq1, Wq2, bq2, Wrel, brel,
           Wqa, bqa, Wka, bka, Wva, bva, Woa, boa,
           gamma, beta, Wh1, bh1, Wh2, bh2, Wh3, bh3):
    q = query_points_coords.astype(jnp.float32)
    c = context_coords.astype(jnp.float32)
    feats = context_features.astype(jnp.float32)
    n = c.shape[0]

    ct = jnp.concatenate(
        [c.T, jnp.full((3, NPAD - n), 100.0, jnp.float32)], axis=1)
    ct = jnp.concatenate([ct, jnp.zeros((5, NPAD), jnp.float32)], axis=0)

    feats_pad = jnp.pad(feats, ((0, NROWPAD - n), (0, 0)))
    coords_pad = jnp.pad(c, ((0, NROWPAD - n), (0, 0)))

    t_table = _prep(feats_pad, coords_pad, Wrel)
    nn_idx = _knn(q, ct)
    g = _gather(nn_idx.reshape(-1), t_table)

    r2 = lambda b: b.reshape(1, -1)
    weights = [Wq1, r2(bq1), Wq2, r2(bq2), Wrel, r2(brel),
               Wqa, r2(bqa), Wka, r2(bka), Wva, r2(bva), Woa, r2(boa),
               r2(gamma), r2(beta), Wh1, r2(bh1), Wh2, r2(bh2), Wh3, r2(bh3)]
    return _dense(q, g, weights)
